# Initial kernel scaffold; baseline (speedup 1.0000x reference)
import jax
import jax.numpy as jnp
from jax.experimental import pallas as pl


def kernel(x_metric, x_alert, edge_index_corr, edge_weight_corr, edge_index_cause, edge_weight_cause, Wr_c0, br_c0, Wroot_c0, Wr_a0, br_a0, Wroot_a0, Wr_c1, br_c1, Wroot_c1, Wr_a1, br_a1, Wroot_a1):
    raise NotImplementedError("write your pallas kernel here")


# speedup vs baseline: 1.0149x; 1.0149x over previous
import functools

import jax
import jax.numpy as jnp
from jax.experimental import pallas as pl


def _leaky(x):
    return jnp.where(x >= 0, x, 0.01 * x)


def _fused_layer_kernel(agg_ref, x_ref, wr_ref, br_ref, wroot_ref, o_ref, *, finite_fix):
    agg = agg_ref[...]
    if finite_fix:
        agg = jnp.where(jnp.isfinite(agg), agg, 0.0)
    acc = jax.lax.dot_general(agg, wr_ref[...], (((1,), (1,)), ((), ())),
                              preferred_element_type=jnp.float32)
    acc += jax.lax.dot_general(x_ref[...], wroot_ref[...], (((1,), (1,)), ((), ())),
                               preferred_element_type=jnp.float32)
    acc += br_ref[...][None, :]
    o_ref[...] = _leaky(acc)


def _fused_layer(agg, x, wr, br, wroot, *, finite_fix=False, block=1000):
    n, d_in = agg.shape
    d_out = wr.shape[0]
    grid = (n + block - 1) // block
    return pl.pallas_call(
        functools.partial(_fused_layer_kernel, finite_fix=finite_fix),
        grid=(grid,),
        in_specs=[
            pl.BlockSpec((block, d_in), lambda i: (i, 0)),
            pl.BlockSpec((block, d_in), lambda i: (i, 0)),
            pl.BlockSpec((d_out, d_in), lambda i: (0, 0)),
            pl.BlockSpec((d_out,), lambda i: (0,)),
            pl.BlockSpec((d_out, d_in), lambda i: (0, 0)),
        ],
        out_specs=pl.BlockSpec((block, d_out), lambda i: (i, 0)),
        out_shape=jax.ShapeDtypeStruct((n, d_out), jnp.float32),
    )(agg, x, wr, br, wroot)


def kernel(x_metric, x_alert, edge_index_corr, edge_weight_corr,
           edge_index_cause, edge_weight_cause,
           Wr_c0, br_c0, Wroot_c0, Wr_a0, br_a0, Wroot_a0,
           Wr_c1, br_c1, Wroot_c1, Wr_a1, br_a1, Wroot_a1):
    n_m = x_metric.shape[0]
    n_a = x_alert.shape[0]

    src_c, dst_c = edge_index_corr[0], edge_index_corr[1]
    src_a, dst_a = edge_index_cause[0], edge_index_cause[1]

    msg_c = x_metric[src_c] * edge_weight_corr[:, None]
    agg_c = jax.ops.segment_sum(msg_c, dst_c, num_segments=n_m)
    m0 = _fused_layer(agg_c, x_metric, Wr_c0, br_c0, Wroot_c0)

    msg_a0 = x_metric[src_a] * edge_weight_cause[:, None]
    agg_a0 = jax.ops.segment_max(msg_a0, dst_a, num_segments=n_a)
    a0 = _fused_layer(agg_a0, x_alert, Wr_a0, br_a0, Wroot_a0, finite_fix=True)

    msg_a1 = m0[src_a] * edge_weight_cause[:, None]
    agg_a1 = jax.ops.segment_max(msg_a1, dst_a, num_segments=n_a)
    a1 = _fused_layer(agg_a1, a0, Wr_a1, br_a1, Wroot_a1, finite_fix=True)
    return a1


# speedup vs baseline: 1.7453x; 1.7197x over previous
import functools

import jax
import jax.numpy as jnp
from jax import lax
from jax.experimental import pallas as pl
from jax.experimental.pallas import tpu as pltpu
from jax.experimental.pallas import tpu_sc as plsc

N_LANES = 16


def _leaky(x):
    return jnp.where(x >= 0, x, 0.01 * x)


def _fused_layer_kernel(agg_ref, x_ref, wr_ref, br_ref, wroot_ref, o_ref, *,
                        finite_fix, n_parts):
    if n_parts == 1:
        agg = agg_ref[0]
    else:
        agg = agg_ref[0] + agg_ref[1]
    if finite_fix:
        agg = jnp.where(jnp.isfinite(agg), agg, 0.0)
    acc = lax.dot_general(agg, wr_ref[...], (((1,), (1,)), ((), ())),
                          preferred_element_type=jnp.float32)
    acc += lax.dot_general(x_ref[...], wroot_ref[...], (((1,), (1,)), ((), ())),
                           preferred_element_type=jnp.float32)
    acc += br_ref[...][None, :]
    o_ref[...] = _leaky(acc)


def _fused_layer(agg, x, wr, br, wroot, *, finite_fix=False, block=1000):
    p, n, d_in = agg.shape
    d_out = wr.shape[0]
    grid = (n + block - 1) // block
    return pl.pallas_call(
        functools.partial(_fused_layer_kernel, finite_fix=finite_fix, n_parts=p),
        grid=(grid,),
        in_specs=[
            pl.BlockSpec((p, block, d_in), lambda i: (0, i, 0)),
            pl.BlockSpec((block, d_in), lambda i: (i, 0)),
            pl.BlockSpec((d_out, d_in), lambda i: (0, 0)),
            pl.BlockSpec((d_out,), lambda i: (0,)),
            pl.BlockSpec((d_out, d_in), lambda i: (0, 0)),
        ],
        out_specs=pl.BlockSpec((block, d_out), lambda i: (i, 0)),
        out_shape=jax.ShapeDtypeStruct((n, d_out), jnp.float32),
    )(agg, x, wr, br, wroot)


def _make_corr_sum(n_rows, d, n_edges):
    NC, NS = 2, 16
    NW = NC * NS
    e_per_w = n_edges // NW
    CH = 80
    n_chunks = e_per_w // CH
    assert e_per_w % CH == 0
    ZR = 128
    rows_per_tile = -(-n_rows // (NS * ZR)) * ZR
    n_pad = rows_per_tile * NS
    mesh = plsc.VectorSubcoreMesh(core_axis_name="c", subcore_axis_name="s")

    @functools.partial(
        pl.kernel, mesh=mesh,
        out_type=jax.ShapeDtypeStruct((NC, n_pad, d), jnp.float32),
        scratch_types=[
            pltpu.VMEM((CH,), jnp.int32),
            pltpu.VMEM((CH,), jnp.int32),
            pltpu.VMEM((CH,), jnp.float32),
            pltpu.VMEM((CH, d), jnp.float32),
            pltpu.VMEM((ZR, d), jnp.float32),
            pltpu.VMEM_SHARED((n_pad, d), jnp.float32),
            pltpu.SemaphoreType.DMA,
        ],
    )
    def corr_sum(src_hbm, dst_hbm, w_hbm, x_hbm, out_hbm,
                 src_v, dst_v, w_v, rows_v, zero_v, acc_sh, sem):
        cid = lax.axis_index("c")
        sid = lax.axis_index("s")
        wid = sid * NC + cid

        zeros16 = jnp.zeros((N_LANES,), jnp.float32)

        def zrow(i, _):
            for j in range(d // N_LANES):
                zero_v[i, pl.ds(j * N_LANES, N_LANES)] = zeros16
            return 0
        lax.fori_loop(0, ZR, zrow, 0)
        for t in range(rows_per_tile // ZR):
            pltpu.sync_copy(zero_v,
                            acc_sh.at[pl.ds(sid * rows_per_tile + t * ZR, ZR)])
        plsc.subcore_barrier()

        def chunk(k, _):
            base = wid * e_per_w + k * CH
            pltpu.sync_copy(src_hbm.at[pl.ds(base, CH)], src_v)
            pltpu.sync_copy(dst_hbm.at[pl.ds(base, CH)], dst_v)
            pltpu.sync_copy(w_hbm.at[pl.ds(base, CH)], w_v)
            pltpu.async_copy(x_hbm.at[src_v], rows_v, sem).wait()

            def rowgrp(g, _):
                w16 = w_v[pl.ds(g * N_LANES, N_LANES)]
                for r in range(N_LANES):
                    i = g * N_LANES + r
                    wb = jnp.full((N_LANES,), w16[r], jnp.float32)
                    for j in range(d // N_LANES):
                        sl = pl.ds(j * N_LANES, N_LANES)
                        rows_v[i, sl] = rows_v[i, sl] * wb
                return 0
            lax.fori_loop(0, CH // N_LANES, rowgrp, 0)
            pltpu.sync_copy(rows_v, acc_sh.at[dst_v], add=True)
            return 0
        lax.fori_loop(0, n_chunks, chunk, 0)
        plsc.subcore_barrier()

        for t in range(rows_per_tile // ZR):
            r0 = sid * rows_per_tile + t * ZR
            pltpu.sync_copy(acc_sh.at[pl.ds(r0, ZR)],
                            out_hbm.at[cid, pl.ds(r0, ZR)])

    return corr_sum


def kernel(x_metric, x_alert, edge_index_corr, edge_weight_corr,
           edge_index_cause, edge_weight_cause,
           Wr_c0, br_c0, Wroot_c0, Wr_a0, br_a0, Wroot_a0,
           Wr_c1, br_c1, Wroot_c1, Wr_a1, br_a1, Wroot_a1):
    n_m, d = x_metric.shape
    n_a = x_alert.shape[0]
    e_c = edge_index_corr.shape[1]

    src_c, dst_c = edge_index_corr[0], edge_index_corr[1]
    src_a, dst_a = edge_index_cause[0], edge_index_cause[1]

    agg_c = _make_corr_sum(n_m, d, e_c)(src_c, dst_c, edge_weight_corr, x_metric)
    m0 = _fused_layer(agg_c[:, :n_m], x_metric, Wr_c0, br_c0, Wroot_c0)

    msg_a0 = x_metric[src_a] * edge_weight_cause[:, None]
    agg_a0 = jax.ops.segment_max(msg_a0, dst_a, num_segments=n_a)
    a0 = _fused_layer(agg_a0[None], x_alert, Wr_a0, br_a0, Wroot_a0,
                      finite_fix=True)

    msg_a1 = m0[src_a] * edge_weight_cause[:, None]
    agg_a1 = jax.ops.segment_max(msg_a1, dst_a, num_segments=n_a)
    a1 = _fused_layer(agg_a1[None], a0, Wr_a1, br_a1, Wroot_a1,
                      finite_fix=True)
    return a1


# speedup vs baseline: 1.7802x; 1.0200x over previous
import functools

import jax
import jax.numpy as jnp
from jax import lax
from jax.experimental import pallas as pl
from jax.experimental.pallas import tpu as pltpu
from jax.experimental.pallas import tpu_sc as plsc

N_LANES = 16


def _leaky(x):
    return jnp.where(x >= 0, x, 0.01 * x)


def _fused_layer_kernel(agg_ref, x_ref, wr_ref, br_ref, wroot_ref, o_ref, *,
                        finite_fix, n_parts):
    if n_parts == 1:
        agg = agg_ref[0]
    else:
        agg = agg_ref[0] + agg_ref[1]
    if finite_fix:
        agg = jnp.where(jnp.isfinite(agg), agg, 0.0)
    acc = lax.dot_general(agg, wr_ref[...], (((1,), (1,)), ((), ())),
                          preferred_element_type=jnp.float32)
    acc += lax.dot_general(x_ref[...], wroot_ref[...], (((1,), (1,)), ((), ())),
                           preferred_element_type=jnp.float32)
    acc += br_ref[...][None, :]
    o_ref[...] = _leaky(acc)


def _fused_layer(agg, x, wr, br, wroot, *, finite_fix=False, block=1000):
    p, n, d_in = agg.shape
    d_out = wr.shape[0]
    grid = (n + block - 1) // block
    return pl.pallas_call(
        functools.partial(_fused_layer_kernel, finite_fix=finite_fix, n_parts=p),
        grid=(grid,),
        in_specs=[
            pl.BlockSpec((p, block, d_in), lambda i: (0, i, 0)),
            pl.BlockSpec((block, d_in), lambda i: (i, 0)),
            pl.BlockSpec((d_out, d_in), lambda i: (0, 0)),
            pl.BlockSpec((d_out,), lambda i: (0,)),
            pl.BlockSpec((d_out, d_in), lambda i: (0, 0)),
        ],
        out_specs=pl.BlockSpec((block, d_out), lambda i: (i, 0)),
        out_shape=jax.ShapeDtypeStruct((n, d_out), jnp.float32),
    )(agg, x, wr, br, wroot)


def _make_corr_sum(n_rows, d, n_edges):
    NC, NS = 2, 16
    NW = NC * NS
    e_per_w = n_edges // NW
    CH = 80
    n_chunks = e_per_w // CH
    assert e_per_w % CH == 0
    ZR = 128
    rows_per_tile = -(-n_rows // (NS * ZR)) * ZR
    n_pad = rows_per_tile * NS
    mesh = plsc.VectorSubcoreMesh(core_axis_name="c", subcore_axis_name="s")

    @functools.partial(
        pl.kernel, mesh=mesh,
        out_type=jax.ShapeDtypeStruct((NC, n_pad, d), jnp.float32),
        scratch_types=[
            pltpu.VMEM((CH,), jnp.int32),
            pltpu.VMEM((CH,), jnp.int32),
            pltpu.VMEM((CH,), jnp.float32),
            pltpu.VMEM((CH, d), jnp.float32),
            pltpu.VMEM((ZR, d), jnp.float32),
            pltpu.VMEM_SHARED((n_pad, d), jnp.float32),
            pltpu.SemaphoreType.DMA,
        ],
    )
    def corr_sum(src_hbm, dst_hbm, w_hbm, x_hbm, out_hbm,
                 src_v, dst_v, w_v, rows_v, zero_v, acc_sh, sem):
        cid = lax.axis_index("c")
        sid = lax.axis_index("s")
        wid = sid * NC + cid

        zeros16 = jnp.zeros((N_LANES,), jnp.float32)

        def zrow(i, _):
            for j in range(d // N_LANES):
                zero_v[i, pl.ds(j * N_LANES, N_LANES)] = zeros16
            return 0
        lax.fori_loop(0, ZR, zrow, 0)
        for t in range(rows_per_tile // ZR):
            pltpu.sync_copy(zero_v,
                            acc_sh.at[pl.ds(sid * rows_per_tile + t * ZR, ZR)])
        plsc.subcore_barrier()

        def chunk(k, _):
            base = wid * e_per_w + k * CH
            pltpu.sync_copy(src_hbm.at[pl.ds(base, CH)], src_v)
            pltpu.sync_copy(dst_hbm.at[pl.ds(base, CH)], dst_v)
            pltpu.sync_copy(w_hbm.at[pl.ds(base, CH)], w_v)
            pltpu.async_copy(x_hbm.at[src_v], rows_v, sem).wait()

            def rowgrp(g, _):
                w16 = w_v[pl.ds(g * N_LANES, N_LANES)]
                for r in range(N_LANES):
                    i = g * N_LANES + r
                    wb = jnp.full((N_LANES,), w16[r], jnp.float32)
                    for j in range(d // N_LANES):
                        sl = pl.ds(j * N_LANES, N_LANES)
                        rows_v[i, sl] = rows_v[i, sl] * wb
                return 0
            lax.fori_loop(0, CH // N_LANES, rowgrp, 0)
            pltpu.sync_copy(rows_v, acc_sh.at[dst_v], add=True)
            return 0
        lax.fori_loop(0, n_chunks, chunk, 0)
        plsc.subcore_barrier()

        for t in range(rows_per_tile // ZR):
            r0 = sid * rows_per_tile + t * ZR
            pltpu.sync_copy(acc_sh.at[pl.ds(r0, ZR)],
                            out_hbm.at[cid, pl.ds(r0, ZR)])

    return corr_sum


def _make_cause_max(n_dst, d, n_edges):
    NC, NS = 2, 16
    NW = NC * NS
    RT = ((-(-n_dst // NW)) + 7) // 8 * 8
    n_pad = RT * NW
    ACC_R = RT + 8
    DUMMY = RT
    CH = 1280
    n_chunks = n_edges // CH
    assert n_edges % CH == 0
    F = 128
    FLUSH_AT = F - N_LANES
    NGRP = F // N_LANES
    mesh = plsc.VectorSubcoreMesh(core_axis_name="c", subcore_axis_name="s")

    @functools.partial(
        pl.kernel, mesh=mesh,
        out_type=(jax.ShapeDtypeStruct((n_pad, d), jnp.float32),
                  jax.ShapeDtypeStruct((n_pad, d), jnp.float32)),
        scratch_types=[
            pltpu.VMEM((CH,), jnp.int32),
            pltpu.VMEM((CH,), jnp.int32),
            pltpu.VMEM((CH,), jnp.float32),
            pltpu.VMEM((F,), jnp.int32),
            pltpu.VMEM((F,), jnp.float32),
            pltpu.VMEM((F,), jnp.int32),
            pltpu.VMEM((F, d), jnp.float32),
            pltpu.VMEM((F, d), jnp.float32),
            pltpu.VMEM((ACC_R, d), jnp.float32),
            pltpu.VMEM((ACC_R, d), jnp.float32),
            pltpu.SemaphoreType.DMA,
            pltpu.SemaphoreType.DMA,
        ],
    )
    def cause_max(src_hbm, dst_hbm, w_hbm, x0_hbm, x1_hbm, out0_hbm, out1_hbm,
                  srcc_v, dstc_v, wc_v, sstag, wstag, dstag,
                  rows0_v, rows1_v, acc0_v, acc1_v, sem0, sem1):
        cid = lax.axis_index("c")
        sid = lax.axis_index("s")
        wid = sid * NC + cid
        lo = wid * RT

        def initrow(i, _):
            ninf16 = jnp.full((N_LANES,), -jnp.inf, jnp.float32)
            for j in range(d // N_LANES):
                sl = pl.ds(j * N_LANES, N_LANES)
                acc0_v[i, sl] = ninf16
                acc1_v[i, sl] = ninf16
            return 0
        lax.fori_loop(0, ACC_R, initrow, 0)

        def dummy_fill(g, _):
            sl = pl.ds(g * N_LANES, N_LANES)
            sstag[sl] = jnp.zeros((N_LANES,), jnp.int32)
            wstag[sl] = jnp.zeros((N_LANES,), jnp.float32)
            dstag[sl] = jnp.full((N_LANES,), DUMMY, jnp.int32)
            return 0
        lax.fori_loop(0, NGRP, dummy_fill, 0)

        def flush():
            cp0 = pltpu.async_copy(x0_hbm.at[sstag], rows0_v, sem0)
            cp1 = pltpu.async_copy(x1_hbm.at[sstag], rows1_v, sem1)
            cp0.wait()
            cp1.wait()

            def grp(g, _):
                w16 = wstag[pl.ds(g * N_LANES, N_LANES)]
                d16 = dstag[pl.ds(g * N_LANES, N_LANES)]
                for r in range(N_LANES):
                    i = g * N_LANES + r
                    wb = jnp.full((N_LANES,), w16[r], jnp.float32)
                    dr = d16[r]
                    for j in range(d // N_LANES):
                        sl = pl.ds(j * N_LANES, N_LANES)
                        acc0_v[dr, sl] = jnp.maximum(acc0_v[dr, sl],
                                                     rows0_v[i, sl] * wb)
                        acc1_v[dr, sl] = jnp.maximum(acc1_v[dr, sl],
                                                     rows1_v[i, sl] * wb)
                return 0
            lax.fori_loop(0, NGRP, grp, 0)

        def grp_scan(g, ptr):
            sl = pl.ds(g * N_LANES, N_LANES)
            d16 = dstc_v[sl]
            s16 = srcc_v[sl]
            w16 = wc_v[sl]
            lane = lax.iota(jnp.int32, N_LANES)
            one = jnp.full((N_LANES,), 1, jnp.int32)
            zero = jnp.full((N_LANES,), 0, jnp.int32)
            lo16 = jnp.full((N_LANES,), lo, jnp.int32)
            hi16 = jnp.full((N_LANES,), lo + RT, jnp.int32)
            m = (d16 >= lo16) & (d16 < hi16)
            pc = jnp.where(m, one, zero)
            for s in (1, 2, 4, 8):
                idx = jnp.maximum(lane - s, 0)
                sh = pc.at[idx].get(mode='promise_in_bounds')
                pc = pc + jnp.where(lane >= s, sh, zero)
            cnt = pc[N_LANES - 1]
            target = lane + one
            pos = zero
            for s in (8, 4, 2, 1):
                probe = pos + jnp.full((N_LANES,), s - 1, jnp.int32)
                v = pc.at[probe].get(mode='promise_in_bounds')
                pos = jnp.where(v < target,
                                pos + jnp.full((N_LANES,), s, jnp.int32), pos)
            cnt16 = jnp.full((N_LANES,), cnt, jnp.int32)
            valid = lane < cnt16
            sg = s16.at[pos].get(mode='promise_in_bounds')
            wg = w16.at[pos].get(mode='promise_in_bounds')
            dg = d16.at[pos].get(mode='promise_in_bounds')
            psl = pl.ds(ptr, N_LANES)
            sstag[psl] = jnp.where(valid, sg, zero)
            wstag[psl] = jnp.where(valid, wg,
                                   jnp.full((N_LANES,), 0.0, jnp.float32))
            dstag[psl] = jnp.where(valid, dg - lo16,
                                   jnp.full((N_LANES,), DUMMY, jnp.int32))
            ptr = ptr + cnt
            do = ptr >= FLUSH_AT

            @pl.when(do)
            def _():
                flush()
            return jnp.where(do, 0, ptr)

        def chunk(k, ptr):
            base = k * CH
            pltpu.sync_copy(src_hbm.at[pl.ds(base, CH)], srcc_v)
            pltpu.sync_copy(dst_hbm.at[pl.ds(base, CH)], dstc_v)
            pltpu.sync_copy(w_hbm.at[pl.ds(base, CH)], wc_v)
            return lax.fori_loop(0, CH // N_LANES, grp_scan, ptr)

        ptr = lax.fori_loop(0, n_chunks, chunk, 0)

        @pl.when(ptr > 0)
        def _():
            flush()

        pltpu.sync_copy(acc0_v.at[pl.ds(0, RT)], out0_hbm.at[pl.ds(lo, RT)])
        pltpu.sync_copy(acc1_v.at[pl.ds(0, RT)], out1_hbm.at[pl.ds(lo, RT)])

    return cause_max


def kernel(x_metric, x_alert, edge_index_corr, edge_weight_corr,
           edge_index_cause, edge_weight_cause,
           Wr_c0, br_c0, Wroot_c0, Wr_a0, br_a0, Wroot_a0,
           Wr_c1, br_c1, Wroot_c1, Wr_a1, br_a1, Wroot_a1):
    n_m, d = x_metric.shape
    n_a = x_alert.shape[0]
    e_c = edge_index_corr.shape[1]

    src_c, dst_c = edge_index_corr[0], edge_index_corr[1]
    src_a, dst_a = edge_index_cause[0], edge_index_cause[1]

    agg_c = _make_corr_sum(n_m, d, e_c)(src_c, dst_c, edge_weight_corr, x_metric)
    m0 = _fused_layer(agg_c[:, :n_m], x_metric, Wr_c0, br_c0, Wroot_c0)

    e_a = edge_index_cause.shape[1]
    agg_a0, agg_a1 = _make_cause_max(n_a, d, e_a)(
        src_a, dst_a, edge_weight_cause, x_metric, m0)
    a0 = _fused_layer(agg_a0[None, :n_a], x_alert, Wr_a0, br_a0, Wroot_a0,
                      finite_fix=True)
    a1 = _fused_layer(agg_a1[None, :n_a], a0, Wr_a1, br_a1, Wroot_a1,
                      finite_fix=True)
    return a1


# speedup vs baseline: 1.8500x; 1.0392x over previous
import functools

import jax
import jax.numpy as jnp
from jax import lax
from jax.experimental import pallas as pl
from jax.experimental.pallas import tpu as pltpu
from jax.experimental.pallas import tpu_sc as plsc

N_LANES = 16


def _leaky(x):
    return jnp.where(x >= 0, x, 0.01 * x)


def _fused_layer_kernel(agg_ref, x_ref, wr_ref, br_ref, wroot_ref, o_ref, *,
                        finite_fix, n_parts):
    if n_parts == 1:
        agg = agg_ref[0]
    else:
        agg = agg_ref[0] + agg_ref[1]
    if finite_fix:
        agg = jnp.where(jnp.isfinite(agg), agg, 0.0)
    acc = lax.dot_general(agg, wr_ref[...], (((1,), (1,)), ((), ())),
                          preferred_element_type=jnp.float32)
    acc += lax.dot_general(x_ref[...], wroot_ref[...], (((1,), (1,)), ((), ())),
                           preferred_element_type=jnp.float32)
    acc += br_ref[...][None, :]
    o_ref[...] = _leaky(acc)


def _fused_layer(agg, x, wr, br, wroot, *, finite_fix=False, block=1000):
    p, n, d_in = agg.shape
    d_out = wr.shape[0]
    grid = (n + block - 1) // block
    return pl.pallas_call(
        functools.partial(_fused_layer_kernel, finite_fix=finite_fix, n_parts=p),
        grid=(grid,),
        in_specs=[
            pl.BlockSpec((p, block, d_in), lambda i: (0, i, 0)),
            pl.BlockSpec((block, d_in), lambda i: (i, 0)),
            pl.BlockSpec((d_out, d_in), lambda i: (0, 0)),
            pl.BlockSpec((d_out,), lambda i: (0,)),
            pl.BlockSpec((d_out, d_in), lambda i: (0, 0)),
        ],
        out_specs=pl.BlockSpec((block, d_out), lambda i: (i, 0)),
        out_shape=jax.ShapeDtypeStruct((n, d_out), jnp.float32),
    )(agg, x, wr, br, wroot)


def _make_corr_sum(n_rows, d, n_edges):
    NC, NS = 2, 16
    NW = NC * NS
    e_per_w = n_edges // NW
    CH = 80
    n_chunks = e_per_w // CH
    assert e_per_w % CH == 0
    ZR = 128
    rows_per_tile = -(-n_rows // (NS * ZR)) * ZR
    n_pad = rows_per_tile * NS
    mesh = plsc.VectorSubcoreMesh(core_axis_name="c", subcore_axis_name="s")

    @functools.partial(
        pl.kernel, mesh=mesh,
        out_type=jax.ShapeDtypeStruct((NC, n_pad, d), jnp.float32),
        scratch_types=[
            pltpu.VMEM((CH,), jnp.int32),
            pltpu.VMEM((CH,), jnp.int32),
            pltpu.VMEM((CH,), jnp.float32),
            pltpu.VMEM((CH, d), jnp.float32),
            pltpu.VMEM((ZR, d), jnp.float32),
            pltpu.VMEM_SHARED((n_pad, d), jnp.float32),
            pltpu.SemaphoreType.DMA,
        ],
    )
    def corr_sum(src_hbm, dst_hbm, w_hbm, x_hbm, out_hbm,
                 src_v, dst_v, w_v, rows_v, zero_v, acc_sh, sem):
        cid = lax.axis_index("c")
        sid = lax.axis_index("s")
        wid = sid * NC + cid

        zeros16 = jnp.zeros((N_LANES,), jnp.float32)

        def zrow(i, _):
            for j in range(d // N_LANES):
                zero_v[i, pl.ds(j * N_LANES, N_LANES)] = zeros16
            return 0
        lax.fori_loop(0, ZR, zrow, 0)
        for t in range(rows_per_tile // ZR):
            pltpu.sync_copy(zero_v,
                            acc_sh.at[pl.ds(sid * rows_per_tile + t * ZR, ZR)])
        plsc.subcore_barrier()

        def chunk(k, _):
            base = wid * e_per_w + k * CH
            pltpu.sync_copy(src_hbm.at[pl.ds(base, CH)], src_v)
            pltpu.sync_copy(dst_hbm.at[pl.ds(base, CH)], dst_v)
            pltpu.sync_copy(w_hbm.at[pl.ds(base, CH)], w_v)
            pltpu.async_copy(x_hbm.at[src_v], rows_v, sem).wait()

            def rowgrp(g, _):
                w16 = w_v[pl.ds(g * N_LANES, N_LANES)]
                for r in range(N_LANES):
                    i = g * N_LANES + r
                    wb = jnp.full((N_LANES,), w16[r], jnp.float32)
                    for j in range(d // N_LANES):
                        sl = pl.ds(j * N_LANES, N_LANES)
                        rows_v[i, sl] = rows_v[i, sl] * wb
                return 0
            lax.fori_loop(0, CH // N_LANES, rowgrp, 0)
            pltpu.sync_copy(rows_v, acc_sh.at[dst_v], add=True)
            return 0
        lax.fori_loop(0, n_chunks, chunk, 0)
        plsc.subcore_barrier()

        for t in range(rows_per_tile // ZR):
            r0 = sid * rows_per_tile + t * ZR
            pltpu.sync_copy(acc_sh.at[pl.ds(r0, ZR)],
                            out_hbm.at[cid, pl.ds(r0, ZR)])

    return corr_sum


def _make_cause_max(n_dst, d2, n_edges):
    NC, NS = 2, 16
    NW = NC * NS
    RT = ((-(-n_dst // NW)) + 7) // 8 * 8
    n_pad = RT * NW
    ACC_R = RT + 8
    DUMMY = RT
    CH = 1280
    n_chunks = n_edges // CH
    assert n_edges % CH == 0
    SS = 2048
    BL = 128
    FLUSH_AT = SS - N_LANES
    mesh = plsc.VectorSubcoreMesh(core_axis_name="c", subcore_axis_name="s")

    @functools.partial(
        pl.kernel, mesh=mesh,
        out_type=jax.ShapeDtypeStruct((n_pad, d2), jnp.float32),
        scratch_types=[
            pltpu.VMEM((CH,), jnp.int32),
            pltpu.VMEM((CH,), jnp.int32),
            pltpu.VMEM((CH,), jnp.float32),
            pltpu.VMEM((SS,), jnp.int32),
            pltpu.VMEM((SS,), jnp.float32),
            pltpu.VMEM((SS,), jnp.int32),
            pltpu.VMEM((BL, d2), jnp.float32),
            pltpu.VMEM((BL, d2), jnp.float32),
            pltpu.VMEM((ACC_R, d2), jnp.float32),
            pltpu.SemaphoreType.DMA,
            pltpu.SemaphoreType.DMA,
        ],
    )
    def cause_max(src_hbm, dst_hbm, w_hbm, x01_hbm, out_hbm,
                  srcc_v, dstc_v, wc_v, sstag, wstag, dstag,
                  rb0, rb1, acc_v, sem0, sem1):
        cid = lax.axis_index("c")
        sid = lax.axis_index("s")
        wid = sid * NC + cid
        lo = wid * RT

        def initrow(i, _):
            ninf16 = jnp.full((N_LANES,), -jnp.inf, jnp.float32)
            for j in range(d2 // N_LANES):
                acc_v[i, pl.ds(j * N_LANES, N_LANES)] = ninf16
            return 0
        lax.fori_loop(0, ACC_R, initrow, 0)

        def dummy_fill(g, _):
            sl = pl.ds(g * N_LANES, N_LANES)
            sstag[sl] = jnp.zeros((N_LANES,), jnp.int32)
            wstag[sl] = jnp.zeros((N_LANES,), jnp.float32)
            dstag[sl] = jnp.full((N_LANES,), DUMMY, jnp.int32)
            return 0
        lax.fori_loop(0, SS // N_LANES, dummy_fill, 0)

        def issue(b, rb, sem):
            return pltpu.async_copy(
                x01_hbm.at[sstag.at[pl.ds(b * BL, BL)]], rb, sem)

        def process(rb, g0):
            def grp(g, _):
                w16 = wstag[pl.ds(g0 * BL + g * N_LANES, N_LANES)]
                d16 = dstag[pl.ds(g0 * BL + g * N_LANES, N_LANES)]
                for r in range(N_LANES):
                    i = g * N_LANES + r
                    wb = jnp.full((N_LANES,), w16[r], jnp.float32)
                    dr = d16[r]
                    for j in range(d2 // N_LANES):
                        sl = pl.ds(j * N_LANES, N_LANES)
                        acc_v[dr, sl] = jnp.maximum(acc_v[dr, sl],
                                                    rb[i, sl] * wb)
                return 0
            lax.fori_loop(0, BL // N_LANES, grp, 0)

        def flush(ptr):
            nb = (ptr + BL - 1) // BL
            issue(0, rb0, sem0)

            def blk(b, _):
                p = b % 2

                @pl.when(p == 0)
                def _():
                    pltpu.make_async_copy(
                        x01_hbm.at[sstag.at[pl.ds(0, BL)]], rb0, sem0).wait()

                    @pl.when(b + 1 < nb)
                    def _():
                        issue(b + 1, rb1, sem1)
                    process(rb0, b)

                @pl.when(p == 1)
                def _():
                    pltpu.make_async_copy(
                        x01_hbm.at[sstag.at[pl.ds(0, BL)]], rb1, sem1).wait()

                    @pl.when(b + 1 < nb)
                    def _():
                        issue(b + 1, rb0, sem0)
                    process(rb1, b)
                return 0
            lax.fori_loop(0, nb, blk, 0)

        def grp_scan(g, ptr):
            sl = pl.ds(g * N_LANES, N_LANES)
            d16 = dstc_v[sl]
            lane = lax.iota(jnp.int32, N_LANES)
            one = jnp.full((N_LANES,), 1, jnp.int32)
            zero = jnp.full((N_LANES,), 0, jnp.int32)
            lo16 = jnp.full((N_LANES,), lo, jnp.int32)
            hi16 = jnp.full((N_LANES,), lo + RT, jnp.int32)
            m = (d16 >= lo16) & (d16 < hi16)
            pc = jnp.where(m, one, zero)
            for st in (1, 2, 4, 8):
                idx = jnp.maximum(lane - st, 0)
                sh = pc.at[idx].get(mode='promise_in_bounds')
                pc = pc + jnp.where(lane >= st, sh, zero)
            cnt = pc[N_LANES - 1]

            @pl.when(cnt > 0)
            def _():
                s16 = srcc_v[sl]
                w16 = wc_v[sl]
                target = lane + one
                pos = zero
                for st in (8, 4, 2, 1):
                    probe = pos + jnp.full((N_LANES,), st - 1, jnp.int32)
                    v = pc.at[probe].get(mode='promise_in_bounds')
                    pos = jnp.where(
                        v < target,
                        pos + jnp.full((N_LANES,), st, jnp.int32), pos)
                cnt16 = jnp.full((N_LANES,), cnt, jnp.int32)
                valid = lane < cnt16
                sg = s16.at[pos].get(mode='promise_in_bounds')
                wg = w16.at[pos].get(mode='promise_in_bounds')
                dg = d16.at[pos].get(mode='promise_in_bounds')
                psl = pl.ds(ptr, N_LANES)
                sstag[psl] = jnp.where(valid, sg, zero)
                wstag[psl] = jnp.where(valid, wg,
                                       jnp.full((N_LANES,), 0.0, jnp.float32))
                dstag[psl] = jnp.where(valid, dg - lo16,
                                       jnp.full((N_LANES,), DUMMY, jnp.int32))
            ptr = ptr + cnt
            do = ptr >= FLUSH_AT

            @pl.when(do)
            def _():
                flush(jnp.int32(SS))
            return jnp.where(do, 0, ptr)

        def chunk(k, ptr):
            base = k * CH
            pltpu.sync_copy(src_hbm.at[pl.ds(base, CH)], srcc_v)
            pltpu.sync_copy(dst_hbm.at[pl.ds(base, CH)], dstc_v)
            pltpu.sync_copy(w_hbm.at[pl.ds(base, CH)], wc_v)
            return lax.fori_loop(0, CH // N_LANES, grp_scan, ptr)

        ptr = lax.fori_loop(0, n_chunks, chunk, 0)

        @pl.when(ptr > 0)
        def _():
            flush(ptr)

        pltpu.sync_copy(acc_v.at[pl.ds(0, RT)], out_hbm.at[pl.ds(lo, RT)])

    return cause_max


def kernel(x_metric, x_alert, edge_index_corr, edge_weight_corr,
           edge_index_cause, edge_weight_cause,
           Wr_c0, br_c0, Wroot_c0, Wr_a0, br_a0, Wroot_a0,
           Wr_c1, br_c1, Wroot_c1, Wr_a1, br_a1, Wroot_a1):
    n_m, d = x_metric.shape
    n_a = x_alert.shape[0]
    e_c = edge_index_corr.shape[1]

    src_c, dst_c = edge_index_corr[0], edge_index_corr[1]
    src_a, dst_a = edge_index_cause[0], edge_index_cause[1]

    agg_c = _make_corr_sum(n_m, d, e_c)(src_c, dst_c, edge_weight_corr, x_metric)
    m0 = _fused_layer(agg_c[:, :n_m], x_metric, Wr_c0, br_c0, Wroot_c0)

    e_a = edge_index_cause.shape[1]
    x01 = jnp.concatenate([x_metric, m0], axis=1)
    out01 = _make_cause_max(n_a, 2 * d, e_a)(
        src_a, dst_a, edge_weight_cause, x01)
    a0 = _fused_layer(out01[None, :n_a, :d], x_alert, Wr_a0, br_a0, Wroot_a0,
                      finite_fix=True)
    a1 = _fused_layer(out01[None, :n_a, d:], a0, Wr_a1, br_a1, Wroot_a1,
                      finite_fix=True)
    return a1


# speedup vs baseline: 1.9650x; 1.0622x over previous
import functools

import jax
import jax.numpy as jnp
from jax import lax
from jax.experimental import pallas as pl
from jax.experimental.pallas import tpu as pltpu
from jax.experimental.pallas import tpu_sc as plsc

N_LANES = 16


def _leaky(x):
    return jnp.where(x >= 0, x, 0.01 * x)


def _fused_layer_kernel(agg_ref, x_ref, wr_ref, br_ref, wroot_ref, o_ref, *,
                        finite_fix, n_parts):
    if n_parts == 1:
        agg = agg_ref[0]
    else:
        agg = agg_ref[0] + agg_ref[1]
    if finite_fix:
        agg = jnp.where(jnp.isfinite(agg), agg, 0.0)
    acc = lax.dot_general(agg, wr_ref[...], (((1,), (1,)), ((), ())),
                          preferred_element_type=jnp.float32)
    acc += lax.dot_general(x_ref[...], wroot_ref[...], (((1,), (1,)), ((), ())),
                           preferred_element_type=jnp.float32)
    acc += br_ref[...][None, :]
    o_ref[...] = _leaky(acc)


def _fused_layer(agg, x, wr, br, wroot, *, finite_fix=False, block=1000):
    p, n, d_in = agg.shape
    d_out = wr.shape[0]
    grid = (n + block - 1) // block
    return pl.pallas_call(
        functools.partial(_fused_layer_kernel, finite_fix=finite_fix, n_parts=p),
        grid=(grid,),
        in_specs=[
            pl.BlockSpec((p, block, d_in), lambda i: (0, i, 0)),
            pl.BlockSpec((block, d_in), lambda i: (i, 0)),
            pl.BlockSpec((d_out, d_in), lambda i: (0, 0)),
            pl.BlockSpec((d_out,), lambda i: (0,)),
            pl.BlockSpec((d_out, d_in), lambda i: (0, 0)),
        ],
        out_specs=pl.BlockSpec((block, d_out), lambda i: (i, 0)),
        out_shape=jax.ShapeDtypeStruct((n, d_out), jnp.float32),
    )(agg, x, wr, br, wroot)


def _make_corr_sum(n_rows, d, n_edges):
    NC, NS = 2, 16
    NW = NC * NS
    e_per_w = n_edges // NW
    CH = 80
    n_chunks = e_per_w // CH
    assert e_per_w % CH == 0
    ZR = 128
    rows_per_tile = -(-n_rows // (NS * ZR)) * ZR
    n_pad = rows_per_tile * NS
    mesh = plsc.VectorSubcoreMesh(core_axis_name="c", subcore_axis_name="s")

    @functools.partial(
        pl.kernel, mesh=mesh,
        out_type=jax.ShapeDtypeStruct((NC, n_pad, d), jnp.float32),
        scratch_types=[
            pltpu.VMEM((e_per_w,), jnp.int32),
            pltpu.VMEM((CH,), jnp.int32),
            pltpu.VMEM((CH,), jnp.int32),
            pltpu.VMEM((CH,), jnp.float32),
            pltpu.VMEM((CH,), jnp.float32),
            pltpu.VMEM((CH, d), jnp.float32),
            pltpu.VMEM((CH, d), jnp.float32),
            pltpu.VMEM_SHARED((n_pad, d), jnp.float32),
            pltpu.SemaphoreType.DMA,
            pltpu.SemaphoreType.DMA,
            pltpu.SemaphoreType.DMA,
            pltpu.SemaphoreType.DMA,
        ],
    )
    def corr_sum(src_hbm, dst_hbm, w_hbm, x_hbm, out_hbm,
                 src_v, db0, db1, wb0, wb1, rb0, rb1, acc_sh,
                 semg0, semg1, sems0, sems1):
        cid = lax.axis_index("c")
        sid = lax.axis_index("s")
        wid = sid * NC + cid

        pltpu.sync_copy(src_hbm.at[pl.ds(wid * e_per_w, e_per_w)], src_v)

        zeros16 = jnp.zeros((N_LANES,), jnp.float32)

        def zrow(i, _):
            for j in range(d // N_LANES):
                rb0[i, pl.ds(j * N_LANES, N_LANES)] = zeros16
            return 0
        lax.fori_loop(0, CH, zrow, 0)
        for t in range(rows_per_tile // CH):
            pltpu.sync_copy(rb0,
                            acc_sh.at[pl.ds(sid * rows_per_tile + t * CH, CH)])
        plsc.subcore_barrier()

        def gissue(k, rb, db, wb, sem):
            pltpu.async_copy(x_hbm.at[src_v.at[pl.ds(k * CH, CH)]], rb, sem)
            pltpu.async_copy(dst_hbm.at[pl.ds(wid * e_per_w + k * CH, CH)],
                             db, sem)
            pltpu.async_copy(w_hbm.at[pl.ds(wid * e_per_w + k * CH, CH)],
                             wb, sem)

        def gwait(rb, db, wb, sem):
            pltpu.make_async_copy(x_hbm.at[src_v.at[pl.ds(0, CH)]], rb,
                                  sem).wait()
            pltpu.make_async_copy(dst_hbm.at[pl.ds(0, CH)], db, sem).wait()
            pltpu.make_async_copy(w_hbm.at[pl.ds(0, CH)], wb, sem).wait()

        def sissue(rb, db, sem):
            pltpu.async_copy(rb, acc_sh.at[db], sem, add=True)

        def swait(rb, db, sem):
            pltpu.make_async_copy(rb, acc_sh.at[db], sem).wait()

        def scale(wbuf, rb):
            def rowgrp(g, _):
                w16 = wbuf[pl.ds(g * N_LANES, N_LANES)]
                for r in range(N_LANES):
                    i = g * N_LANES + r
                    wb = jnp.full((N_LANES,), w16[r], jnp.float32)
                    for j in range(d // N_LANES):
                        sl = pl.ds(j * N_LANES, N_LANES)
                        rb[i, sl] = rb[i, sl] * wb
                return 0
            lax.fori_loop(0, CH // N_LANES, rowgrp, 0)

        gissue(0, rb0, db0, wb0, semg0)

        def chunk(k, _):
            @pl.when(k % 2 == 0)
            def _():
                gwait(rb0, db0, wb0, semg0)

                @pl.when(k >= 1)
                def _():
                    swait(rb1, db1, sems1)

                @pl.when(k + 1 < n_chunks)
                def _():
                    gissue(k + 1, rb1, db1, wb1, semg1)
                scale(wb0, rb0)
                sissue(rb0, db0, sems0)

            @pl.when(k % 2 == 1)
            def _():
                gwait(rb1, db1, wb1, semg1)
                swait(rb0, db0, sems0)

                @pl.when(k + 1 < n_chunks)
                def _():
                    gissue(k + 1, rb0, db0, wb0, semg0)
                scale(wb1, rb1)
                sissue(rb1, db1, sems1)
            return 0
        lax.fori_loop(0, n_chunks, chunk, 0)
        if (n_chunks - 1) % 2 == 0:
            swait(rb0, db0, sems0)
        else:
            swait(rb1, db1, sems1)
        plsc.subcore_barrier()

        for t in range(rows_per_tile // ZR):
            r0 = sid * rows_per_tile + t * ZR
            pltpu.sync_copy(acc_sh.at[pl.ds(r0, ZR)],
                            out_hbm.at[cid, pl.ds(r0, ZR)])

    return corr_sum


def _make_cause_max(n_dst, d2, n_edges):
    NC, NS = 2, 16
    NW = NC * NS
    RT = ((-(-n_dst // NW)) + 7) // 8 * 8
    n_pad = RT * NW
    ACC_R = RT + 8
    DUMMY = RT
    CH = 1280
    n_chunks = n_edges // CH
    assert n_edges % CH == 0
    SS = 2048
    BL = 128
    FLUSH_AT = SS - 128
    mesh = plsc.VectorSubcoreMesh(core_axis_name="c", subcore_axis_name="s")

    @functools.partial(
        pl.kernel, mesh=mesh,
        out_type=jax.ShapeDtypeStruct((n_pad, d2), jnp.float32),
        scratch_types=[
            pltpu.VMEM((CH,), jnp.int32),
            pltpu.VMEM((CH,), jnp.int32),
            pltpu.VMEM((CH,), jnp.float32),
            pltpu.VMEM((SS,), jnp.int32),
            pltpu.VMEM((SS,), jnp.float32),
            pltpu.VMEM((SS,), jnp.int32),
            pltpu.VMEM((BL, d2), jnp.float32),
            pltpu.VMEM((BL, d2), jnp.float32),
            pltpu.VMEM((ACC_R, d2), jnp.float32),
            pltpu.SemaphoreType.DMA,
            pltpu.SemaphoreType.DMA,
        ],
    )
    def cause_max(src_hbm, dst_hbm, w_hbm, x01_hbm, out_hbm,
                  srcc_v, dstc_v, wc_v, sstag, wstag, dstag,
                  rb0, rb1, acc_v, sem0, sem1):
        cid = lax.axis_index("c")
        sid = lax.axis_index("s")
        wid = sid * NC + cid
        lo = wid * RT

        def initrow(i, _):
            ninf16 = jnp.full((N_LANES,), -jnp.inf, jnp.float32)
            for j in range(d2 // N_LANES):
                acc_v[i, pl.ds(j * N_LANES, N_LANES)] = ninf16
            return 0
        lax.fori_loop(0, ACC_R, initrow, 0)

        def dummy_fill(g, _):
            sl = pl.ds(g * N_LANES, N_LANES)
            sstag[sl] = jnp.zeros((N_LANES,), jnp.int32)
            wstag[sl] = jnp.zeros((N_LANES,), jnp.float32)
            dstag[sl] = jnp.full((N_LANES,), DUMMY, jnp.int32)
            return 0
        lax.fori_loop(0, SS // N_LANES, dummy_fill, 0)

        def issue(b, rb, sem):
            return pltpu.async_copy(
                x01_hbm.at[sstag.at[pl.ds(b * BL, BL)]], rb, sem)

        def process(rb, g0):
            def grp(g, _):
                w16 = wstag[pl.ds(g0 * BL + g * N_LANES, N_LANES)]
                d16 = dstag[pl.ds(g0 * BL + g * N_LANES, N_LANES)]
                for r in range(N_LANES):
                    i = g * N_LANES + r
                    wb = jnp.full((N_LANES,), w16[r], jnp.float32)
                    dr = d16[r]
                    for j in range(d2 // N_LANES):
                        sl = pl.ds(j * N_LANES, N_LANES)
                        acc_v[dr, sl] = jnp.maximum(acc_v[dr, sl],
                                                    rb[i, sl] * wb)
                return 0
            lax.fori_loop(0, BL // N_LANES, grp, 0)

        def flush(ptr):
            nb = (ptr + BL - 1) // BL
            issue(0, rb0, sem0)

            def blk(b, _):
                p = b % 2

                @pl.when(p == 0)
                def _():
                    pltpu.make_async_copy(
                        x01_hbm.at[sstag.at[pl.ds(0, BL)]], rb0, sem0).wait()

                    @pl.when(b + 1 < nb)
                    def _():
                        issue(b + 1, rb1, sem1)
                    process(rb0, b)

                @pl.when(p == 1)
                def _():
                    pltpu.make_async_copy(
                        x01_hbm.at[sstag.at[pl.ds(0, BL)]], rb1, sem1).wait()

                    @pl.when(b + 1 < nb)
                    def _():
                        issue(b + 1, rb0, sem0)
                    process(rb1, b)
                return 0
            lax.fori_loop(0, nb, blk, 0)

        def grp_scan(g, ptr):
            sl = pl.ds(g * N_LANES, N_LANES)
            d16 = dstc_v[sl]
            lane = lax.iota(jnp.int32, N_LANES)
            one = jnp.full((N_LANES,), 1, jnp.int32)
            zero = jnp.full((N_LANES,), 0, jnp.int32)
            lo16 = jnp.full((N_LANES,), lo, jnp.int32)
            hi16 = jnp.full((N_LANES,), lo + RT, jnp.int32)
            m = (d16 >= lo16) & (d16 < hi16)
            pc = jnp.where(m, one, zero)
            for st in (1, 2, 4, 8):
                idx = jnp.maximum(lane - st, 0)
                sh = pc.at[idx].get(mode='promise_in_bounds')
                pc = pc + jnp.where(lane >= st, sh, zero)
            cnt = pc[N_LANES - 1]

            @pl.when(cnt > 0)
            def _():
                s16 = srcc_v[sl]
                w16 = wc_v[sl]
                target = lane + one
                pos = zero
                for st in (8, 4, 2, 1):
                    probe = pos + jnp.full((N_LANES,), st - 1, jnp.int32)
                    v = pc.at[probe].get(mode='promise_in_bounds')
                    pos = jnp.where(
                        v < target,
                        pos + jnp.full((N_LANES,), st, jnp.int32), pos)
                cnt16 = jnp.full((N_LANES,), cnt, jnp.int32)
                valid = lane < cnt16
                sg = s16.at[pos].get(mode='promise_in_bounds')
                wg = w16.at[pos].get(mode='promise_in_bounds')
                dg = d16.at[pos].get(mode='promise_in_bounds')
                psl = pl.ds(ptr, N_LANES)
                sstag[psl] = jnp.where(valid, sg, zero)
                wstag[psl] = jnp.where(valid, wg,
                                       jnp.full((N_LANES,), 0.0, jnp.float32))
                dstag[psl] = jnp.where(valid, dg - lo16,
                                       jnp.full((N_LANES,), DUMMY, jnp.int32))
            return ptr + cnt

        def subchunk(t, ptr):
            def gs(g, p):
                return grp_scan(t * 8 + g, p)
            ptr = lax.fori_loop(0, 8, gs, ptr, unroll=8)
            do = ptr >= FLUSH_AT

            @pl.when(do)
            def _():
                flush(ptr)
            return jnp.where(do, 0, ptr)

        def chunk(k, ptr):
            base = k * CH
            pltpu.sync_copy(src_hbm.at[pl.ds(base, CH)], srcc_v)
            pltpu.sync_copy(dst_hbm.at[pl.ds(base, CH)], dstc_v)
            pltpu.sync_copy(w_hbm.at[pl.ds(base, CH)], wc_v)
            return lax.fori_loop(0, CH // 128, subchunk, ptr)

        ptr = lax.fori_loop(0, n_chunks, chunk, 0)

        @pl.when(ptr > 0)
        def _():
            flush(ptr)

        pltpu.sync_copy(acc_v.at[pl.ds(0, RT)], out_hbm.at[pl.ds(lo, RT)])

    return cause_max


def kernel(x_metric, x_alert, edge_index_corr, edge_weight_corr,
           edge_index_cause, edge_weight_cause,
           Wr_c0, br_c0, Wroot_c0, Wr_a0, br_a0, Wroot_a0,
           Wr_c1, br_c1, Wroot_c1, Wr_a1, br_a1, Wroot_a1):
    n_m, d = x_metric.shape
    n_a = x_alert.shape[0]
    e_c = edge_index_corr.shape[1]

    src_c, dst_c = edge_index_corr[0], edge_index_corr[1]
    src_a, dst_a = edge_index_cause[0], edge_index_cause[1]

    agg_c = _make_corr_sum(n_m, d, e_c)(src_c, dst_c, edge_weight_corr,
                                        x_metric)
    m0 = _fused_layer(agg_c[:, :n_m], x_metric, Wr_c0, br_c0, Wroot_c0)

    e_a = edge_index_cause.shape[1]
    x01 = jnp.concatenate([x_metric, m0], axis=1)
    out01 = _make_cause_max(n_a, 2 * d, e_a)(
        src_a, dst_a, edge_weight_cause, x01)
    a0 = _fused_layer(out01[None, :n_a, :d], x_alert, Wr_a0, br_a0, Wroot_a0,
                      finite_fix=True)
    a1 = _fused_layer(out01[None, :n_a, d:], a0, Wr_a1, br_a1, Wroot_a1,
                      finite_fix=True)
    return a1


# speedup vs baseline: 2.2133x; 1.1264x over previous
import functools

import jax
import jax.numpy as jnp
from jax import lax
from jax.experimental import pallas as pl
from jax.experimental.pallas import tpu as pltpu
from jax.experimental.pallas import tpu_sc as plsc

N_LANES = 16


def _leaky(x):
    return jnp.where(x >= 0, x, 0.01 * x)


def _fused_layer_kernel(agg_ref, x_ref, wr_ref, br_ref, wroot_ref, o_ref, *,
                        finite_fix, n_parts):
    if n_parts == 1:
        agg = agg_ref[0]
    else:
        agg = agg_ref[0] + agg_ref[1]
    if finite_fix:
        agg = jnp.where(jnp.isfinite(agg), agg, 0.0)
    acc = lax.dot_general(agg, wr_ref[...], (((1,), (1,)), ((), ())),
                          preferred_element_type=jnp.float32)
    acc += lax.dot_general(x_ref[...], wroot_ref[...], (((1,), (1,)), ((), ())),
                           preferred_element_type=jnp.float32)
    acc += br_ref[...][None, :]
    o_ref[...] = _leaky(acc)


def _fused_layer(agg, x, wr, br, wroot, *, finite_fix=False, block=1000):
    p, n, d_in = agg.shape
    d_out = wr.shape[0]
    grid = (n + block - 1) // block
    return pl.pallas_call(
        functools.partial(_fused_layer_kernel, finite_fix=finite_fix, n_parts=p),
        grid=(grid,),
        in_specs=[
            pl.BlockSpec((p, block, d_in), lambda i: (0, i, 0)),
            pl.BlockSpec((block, d_in), lambda i: (i, 0)),
            pl.BlockSpec((d_out, d_in), lambda i: (0, 0)),
            pl.BlockSpec((d_out,), lambda i: (0,)),
            pl.BlockSpec((d_out, d_in), lambda i: (0, 0)),
        ],
        out_specs=pl.BlockSpec((block, d_out), lambda i: (i, 0)),
        out_shape=jax.ShapeDtypeStruct((n, d_out), jnp.float32),
    )(agg, x, wr, br, wroot)


def _make_corr_sum(n_rows, d, n_edges):
    NC, NS = 2, 16
    NW = NC * NS
    e_per_w = n_edges // NW
    CH = 80
    n_chunks = e_per_w // CH
    assert e_per_w % CH == 0
    ZR = 128
    rows_per_tile = -(-n_rows // (NS * ZR)) * ZR
    n_pad = rows_per_tile * NS
    mesh = plsc.VectorSubcoreMesh(core_axis_name="c", subcore_axis_name="s")

    @functools.partial(
        pl.kernel, mesh=mesh,
        out_type=jax.ShapeDtypeStruct((NC, n_pad, d), jnp.float32),
        scratch_types=[
            pltpu.VMEM((e_per_w,), jnp.int32),
            pltpu.VMEM((CH,), jnp.int32),
            pltpu.VMEM((CH,), jnp.int32),
            pltpu.VMEM((CH,), jnp.float32),
            pltpu.VMEM((CH,), jnp.float32),
            pltpu.VMEM((CH, d), jnp.float32),
            pltpu.VMEM((CH, d), jnp.float32),
            pltpu.VMEM_SHARED((n_pad, d), jnp.float32),
            pltpu.SemaphoreType.DMA,
            pltpu.SemaphoreType.DMA,
            pltpu.SemaphoreType.DMA,
            pltpu.SemaphoreType.DMA,
        ],
    )
    def corr_sum(src_hbm, dst_hbm, w_hbm, x_hbm, out_hbm,
                 src_v, db0, db1, wb0, wb1, rb0, rb1, acc_sh,
                 semg0, semg1, sems0, sems1):
        cid = lax.axis_index("c")
        sid = lax.axis_index("s")
        wid = sid * NC + cid

        pltpu.sync_copy(src_hbm.at[pl.ds(wid * e_per_w, e_per_w)], src_v)

        zeros16 = jnp.zeros((N_LANES,), jnp.float32)

        def zrow(i, _):
            for j in range(d // N_LANES):
                rb0[i, pl.ds(j * N_LANES, N_LANES)] = zeros16
            return 0
        lax.fori_loop(0, CH, zrow, 0)
        for t in range(rows_per_tile // CH):
            pltpu.sync_copy(rb0,
                            acc_sh.at[pl.ds(sid * rows_per_tile + t * CH, CH)])
        plsc.subcore_barrier()

        def gissue(k, rb, db, wb, sem):
            pltpu.async_copy(x_hbm.at[src_v.at[pl.ds(k * CH, CH)]], rb, sem)
            pltpu.async_copy(dst_hbm.at[pl.ds(wid * e_per_w + k * CH, CH)],
                             db, sem)
            pltpu.async_copy(w_hbm.at[pl.ds(wid * e_per_w + k * CH, CH)],
                             wb, sem)

        def gwait(rb, db, wb, sem):
            pltpu.make_async_copy(x_hbm.at[src_v.at[pl.ds(0, CH)]], rb,
                                  sem).wait()
            pltpu.make_async_copy(dst_hbm.at[pl.ds(0, CH)], db, sem).wait()
            pltpu.make_async_copy(w_hbm.at[pl.ds(0, CH)], wb, sem).wait()

        def sissue(rb, db, sem):
            pltpu.async_copy(rb, acc_sh.at[db], sem, add=True)

        def swait(rb, db, sem):
            pltpu.make_async_copy(rb, acc_sh.at[db], sem).wait()

        def scale(wbuf, rb):
            def rowgrp(g, _):
                w16 = wbuf[pl.ds(g * N_LANES, N_LANES)]
                for r in range(N_LANES):
                    i = g * N_LANES + r
                    wb = jnp.full((N_LANES,), w16[r], jnp.float32)
                    for j in range(d // N_LANES):
                        sl = pl.ds(j * N_LANES, N_LANES)
                        rb[i, sl] = rb[i, sl] * wb
                return 0
            lax.fori_loop(0, CH // N_LANES, rowgrp, 0)

        gissue(0, rb0, db0, wb0, semg0)

        def chunk(k, _):
            @pl.when(k % 2 == 0)
            def _():
                gwait(rb0, db0, wb0, semg0)

                @pl.when(k >= 1)
                def _():
                    swait(rb1, db1, sems1)

                @pl.when(k + 1 < n_chunks)
                def _():
                    gissue(k + 1, rb1, db1, wb1, semg1)
                scale(wb0, rb0)
                sissue(rb0, db0, sems0)

            @pl.when(k % 2 == 1)
            def _():
                gwait(rb1, db1, wb1, semg1)
                swait(rb0, db0, sems0)

                @pl.when(k + 1 < n_chunks)
                def _():
                    gissue(k + 1, rb0, db0, wb0, semg0)
                scale(wb1, rb1)
                sissue(rb1, db1, sems1)
            return 0
        lax.fori_loop(0, n_chunks, chunk, 0)
        if (n_chunks - 1) % 2 == 0:
            swait(rb0, db0, sems0)
        else:
            swait(rb1, db1, sems1)
        plsc.subcore_barrier()

        for t in range(rows_per_tile // ZR):
            r0 = sid * rows_per_tile + t * ZR
            pltpu.sync_copy(acc_sh.at[pl.ds(r0, ZR)],
                            out_hbm.at[cid, pl.ds(r0, ZR)])

    return corr_sum


def _make_cause_max(n_dst, d2, n_edges):
    NC, NS = 2, 16
    NW = NC * NS
    RT = ((-(-n_dst // NW)) + 7) // 8 * 8
    n_pad = RT * NW
    ACC_R = RT + 8
    DUMMY = RT
    CH = 3200
    n_chunks = n_edges // CH
    assert n_edges % CH == 0
    SS = 2016
    BL = 96
    FLUSH_AT = SS - 128
    mesh = plsc.VectorSubcoreMesh(core_axis_name="c", subcore_axis_name="s")

    @functools.partial(
        pl.kernel, mesh=mesh,
        out_type=jax.ShapeDtypeStruct((n_pad, d2), jnp.float32),
        scratch_types=[
            pltpu.VMEM((CH,), jnp.int32),
            pltpu.VMEM((CH,), jnp.int32),
            pltpu.VMEM((CH,), jnp.float32),
            pltpu.VMEM((SS,), jnp.int32),
            pltpu.VMEM((SS,), jnp.float32),
            pltpu.VMEM((SS,), jnp.int32),
            pltpu.VMEM((BL, d2), jnp.float32),
            pltpu.VMEM((BL, d2), jnp.float32),
            pltpu.VMEM((ACC_R, d2), jnp.float32),
            pltpu.SemaphoreType.DMA,
            pltpu.SemaphoreType.DMA,
        ],
    )
    def cause_max(src_hbm, dst_hbm, w_hbm, x01_hbm, out_hbm,
                  srcc_v, dstc_v, wc_v, sstag, wstag, dstag,
                  rb0, rb1, acc_v, sem0, sem1):
        cid = lax.axis_index("c")
        sid = lax.axis_index("s")
        wid = sid * NC + cid
        lo = wid * RT

        def initrow(i, _):
            ninf16 = jnp.full((N_LANES,), -jnp.inf, jnp.float32)
            for j in range(d2 // N_LANES):
                acc_v[i, pl.ds(j * N_LANES, N_LANES)] = ninf16
            return 0
        lax.fori_loop(0, ACC_R, initrow, 0)

        def dummy_fill(g, _):
            sl = pl.ds(g * N_LANES, N_LANES)
            sstag[sl] = jnp.zeros((N_LANES,), jnp.int32)
            wstag[sl] = jnp.zeros((N_LANES,), jnp.float32)
            dstag[sl] = jnp.full((N_LANES,), DUMMY, jnp.int32)
            return 0
        lax.fori_loop(0, SS // N_LANES, dummy_fill, 0)

        def issue(b, rb, sem):
            return pltpu.async_copy(
                x01_hbm.at[sstag.at[pl.ds(b * BL, BL)]], rb, sem)

        def process(rb, g0):
            def grp(g, _):
                w16 = wstag[pl.ds(g0 * BL + g * N_LANES, N_LANES)]
                d16 = dstag[pl.ds(g0 * BL + g * N_LANES, N_LANES)]
                for r in range(N_LANES):
                    i = g * N_LANES + r
                    wb = jnp.full((N_LANES,), w16[r], jnp.float32)
                    dr = d16[r]
                    for j in range(d2 // N_LANES):
                        sl = pl.ds(j * N_LANES, N_LANES)
                        acc_v[dr, sl] = jnp.maximum(acc_v[dr, sl],
                                                    rb[i, sl] * wb)
                return 0
            lax.fori_loop(0, BL // N_LANES, grp, 0)

        def flush(ptr):
            nb = (ptr + BL - 1) // BL
            issue(0, rb0, sem0)

            def blk(b, _):
                p = b % 2

                @pl.when(p == 0)
                def _():
                    pltpu.make_async_copy(
                        x01_hbm.at[sstag.at[pl.ds(0, BL)]], rb0, sem0).wait()

                    @pl.when(b + 1 < nb)
                    def _():
                        issue(b + 1, rb1, sem1)
                    process(rb0, b)

                @pl.when(p == 1)
                def _():
                    pltpu.make_async_copy(
                        x01_hbm.at[sstag.at[pl.ds(0, BL)]], rb1, sem1).wait()

                    @pl.when(b + 1 < nb)
                    def _():
                        issue(b + 1, rb0, sem0)
                    process(rb1, b)
                return 0
            lax.fori_loop(0, nb, blk, 0)

        def grp_scan(g, ptr):
            sl = pl.ds(g * N_LANES, N_LANES)
            d16 = dstc_v[sl]
            lane = lax.iota(jnp.int32, N_LANES)
            one = jnp.full((N_LANES,), 1, jnp.int32)
            zero = jnp.full((N_LANES,), 0, jnp.int32)
            lo16 = jnp.full((N_LANES,), lo, jnp.int32)
            hi16 = jnp.full((N_LANES,), lo + RT, jnp.int32)
            m = (d16 >= lo16) & (d16 < hi16)
            pc = jnp.where(m, one, zero)
            for st in (1, 2, 4, 8):
                idx = jnp.maximum(lane - st, 0)
                sh = pc.at[idx].get(mode='promise_in_bounds')
                pc = pc + jnp.where(lane >= st, sh, zero)
            cnt = pc[N_LANES - 1]

            @pl.when(cnt > 0)
            def _():
                s16 = srcc_v[sl]
                w16 = wc_v[sl]
                target = lane + one
                pos = zero
                for st in (8, 4, 2, 1):
                    probe = pos + jnp.full((N_LANES,), st - 1, jnp.int32)
                    v = pc.at[probe].get(mode='promise_in_bounds')
                    pos = jnp.where(
                        v < target,
                        pos + jnp.full((N_LANES,), st, jnp.int32), pos)
                cnt16 = jnp.full((N_LANES,), cnt, jnp.int32)
                valid = lane < cnt16
                sg = s16.at[pos].get(mode='promise_in_bounds')
                wg = w16.at[pos].get(mode='promise_in_bounds')
                dg = d16.at[pos].get(mode='promise_in_bounds')
                psl = pl.ds(ptr, N_LANES)
                sstag[psl] = jnp.where(valid, sg, zero)
                wstag[psl] = jnp.where(valid, wg,
                                       jnp.full((N_LANES,), 0.0, jnp.float32))
                dstag[psl] = jnp.where(valid, dg - lo16,
                                       jnp.full((N_LANES,), DUMMY, jnp.int32))
            return ptr + cnt

        def subchunk(t, ptr):
            def gs(g, p):
                return grp_scan(t * 8 + g, p)
            ptr = lax.fori_loop(0, 8, gs, ptr, unroll=8)
            do = ptr >= FLUSH_AT

            @pl.when(do)
            def _():
                flush(ptr)
            return jnp.where(do, 0, ptr)

        def chunk(k, ptr):
            base = k * CH
            pltpu.sync_copy(src_hbm.at[pl.ds(base, CH)], srcc_v)
            pltpu.sync_copy(dst_hbm.at[pl.ds(base, CH)], dstc_v)
            pltpu.sync_copy(w_hbm.at[pl.ds(base, CH)], wc_v)
            return lax.fori_loop(0, CH // 128, subchunk, ptr)

        ptr = lax.fori_loop(0, n_chunks, chunk, 0)

        @pl.when(ptr > 0)
        def _():
            flush(ptr)

        pltpu.sync_copy(acc_v.at[pl.ds(0, RT)], out_hbm.at[pl.ds(lo, RT)])

    return cause_max


def kernel(x_metric, x_alert, edge_index_corr, edge_weight_corr,
           edge_index_cause, edge_weight_cause,
           Wr_c0, br_c0, Wroot_c0, Wr_a0, br_a0, Wroot_a0,
           Wr_c1, br_c1, Wroot_c1, Wr_a1, br_a1, Wroot_a1):
    n_m, d = x_metric.shape
    n_a = x_alert.shape[0]
    e_c = edge_index_corr.shape[1]

    src_c, dst_c = edge_index_corr[0], edge_index_corr[1]
    src_a, dst_a = edge_index_cause[0], edge_index_cause[1]

    agg_c = _make_corr_sum(n_m, d, e_c)(src_c, dst_c, edge_weight_corr,
                                        x_metric)
    m0 = _fused_layer(agg_c[:, :n_m], x_metric, Wr_c0, br_c0, Wroot_c0)

    e_a = edge_index_cause.shape[1]
    x01 = jnp.concatenate([x_metric, m0], axis=1)
    out01 = _make_cause_max(n_a, 2 * d, e_a)(
        src_a, dst_a, edge_weight_cause, x01)
    a0 = _fused_layer(out01[None, :n_a, :d], x_alert, Wr_a0, br_a0, Wroot_a0,
                      finite_fix=True)
    a1 = _fused_layer(out01[None, :n_a, d:], a0, Wr_a1, br_a1, Wroot_a1,
                      finite_fix=True)
    return a1


# speedup vs baseline: 2.3274x; 1.0516x over previous
import functools

import jax
import jax.numpy as jnp
from jax import lax
from jax.experimental import pallas as pl
from jax.experimental.pallas import tpu as pltpu
from jax.experimental.pallas import tpu_sc as plsc

N_LANES = 16


def _leaky(x):
    return jnp.where(x >= 0, x, 0.01 * x)


def _fused_layer_kernel(agg_ref, x_ref, wr_ref, br_ref, wroot_ref, o_ref, *,
                        finite_fix, n_parts):
    if n_parts == 1:
        agg = agg_ref[0]
    else:
        agg = agg_ref[0] + agg_ref[1]
    if finite_fix:
        agg = jnp.where(jnp.isfinite(agg), agg, 0.0)
    acc = lax.dot_general(agg, wr_ref[...], (((1,), (1,)), ((), ())),
                          preferred_element_type=jnp.float32)
    acc += lax.dot_general(x_ref[...], wroot_ref[...], (((1,), (1,)), ((), ())),
                           preferred_element_type=jnp.float32)
    acc += br_ref[...][None, :]
    o_ref[...] = _leaky(acc)


def _fused_layer(agg, x, wr, br, wroot, *, finite_fix=False, block=1000):
    p, n, d_in = agg.shape
    d_out = wr.shape[0]
    grid = (n + block - 1) // block
    return pl.pallas_call(
        functools.partial(_fused_layer_kernel, finite_fix=finite_fix, n_parts=p),
        grid=(grid,),
        in_specs=[
            pl.BlockSpec((p, block, d_in), lambda i: (0, i, 0)),
            pl.BlockSpec((block, d_in), lambda i: (i, 0)),
            pl.BlockSpec((d_out, d_in), lambda i: (0, 0)),
            pl.BlockSpec((d_out,), lambda i: (0,)),
            pl.BlockSpec((d_out, d_in), lambda i: (0, 0)),
        ],
        out_specs=pl.BlockSpec((block, d_out), lambda i: (i, 0)),
        out_shape=jax.ShapeDtypeStruct((n, d_out), jnp.float32),
    )(agg, x, wr, br, wroot)


def _make_corr_sum(n_rows, d, n_edges):
    NC, NS = 2, 16
    NW = NC * NS
    e_per_w = n_edges // NW
    CH = 80
    n_chunks = e_per_w // CH
    assert e_per_w % CH == 0
    ZR = 128
    rows_per_tile = -(-n_rows // (NS * ZR)) * ZR
    n_pad = rows_per_tile * NS
    mesh = plsc.VectorSubcoreMesh(core_axis_name="c", subcore_axis_name="s")

    @functools.partial(
        pl.kernel, mesh=mesh,
        out_type=jax.ShapeDtypeStruct((NC, n_pad, d), jnp.float32),
        scratch_types=[
            pltpu.VMEM((e_per_w,), jnp.int32),
            pltpu.VMEM((CH,), jnp.int32),
            pltpu.VMEM((CH,), jnp.int32),
            pltpu.VMEM((CH,), jnp.float32),
            pltpu.VMEM((CH,), jnp.float32),
            pltpu.VMEM((CH, d), jnp.float32),
            pltpu.VMEM((CH, d), jnp.float32),
            pltpu.VMEM_SHARED((n_pad, d), jnp.float32),
            pltpu.SemaphoreType.DMA,
            pltpu.SemaphoreType.DMA,
            pltpu.SemaphoreType.DMA,
            pltpu.SemaphoreType.DMA,
        ],
    )
    def corr_sum(src_hbm, dst_hbm, w_hbm, x_hbm, out_hbm,
                 src_v, db0, db1, wb0, wb1, rb0, rb1, acc_sh,
                 semg0, semg1, sems0, sems1):
        cid = lax.axis_index("c")
        sid = lax.axis_index("s")
        wid = sid * NC + cid

        pltpu.sync_copy(src_hbm.at[pl.ds(wid * e_per_w, e_per_w)], src_v)

        zeros16 = jnp.zeros((N_LANES,), jnp.float32)

        def zrow(i, _):
            for j in range(d // N_LANES):
                rb0[i, pl.ds(j * N_LANES, N_LANES)] = zeros16
            return 0
        lax.fori_loop(0, CH, zrow, 0)
        for t in range(rows_per_tile // CH):
            pltpu.sync_copy(rb0,
                            acc_sh.at[pl.ds(sid * rows_per_tile + t * CH, CH)])
        plsc.subcore_barrier()

        def gissue(k, rb, db, wb, sem):
            pltpu.async_copy(x_hbm.at[src_v.at[pl.ds(k * CH, CH)]], rb, sem)
            pltpu.async_copy(dst_hbm.at[pl.ds(wid * e_per_w + k * CH, CH)],
                             db, sem)
            pltpu.async_copy(w_hbm.at[pl.ds(wid * e_per_w + k * CH, CH)],
                             wb, sem)

        def gwait(rb, db, wb, sem):
            pltpu.make_async_copy(x_hbm.at[src_v.at[pl.ds(0, CH)]], rb,
                                  sem).wait()
            pltpu.make_async_copy(dst_hbm.at[pl.ds(0, CH)], db, sem).wait()
            pltpu.make_async_copy(w_hbm.at[pl.ds(0, CH)], wb, sem).wait()

        def sissue(rb, db, sem):
            pltpu.async_copy(rb, acc_sh.at[db], sem, add=True)

        def swait(rb, db, sem):
            pltpu.make_async_copy(rb, acc_sh.at[db], sem).wait()

        def scale(wbuf, rb):
            def rowgrp(g, _):
                w16 = wbuf[pl.ds(g * N_LANES, N_LANES)]
                for r in range(N_LANES):
                    i = g * N_LANES + r
                    wb = jnp.full((N_LANES,), w16[r], jnp.float32)
                    for j in range(d // N_LANES):
                        sl = pl.ds(j * N_LANES, N_LANES)
                        rb[i, sl] = rb[i, sl] * wb
                return 0
            lax.fori_loop(0, CH // N_LANES, rowgrp, 0)

        gissue(0, rb0, db0, wb0, semg0)

        def chunk(k, _):
            @pl.when(k % 2 == 0)
            def _():
                gwait(rb0, db0, wb0, semg0)

                @pl.when(k >= 1)
                def _():
                    swait(rb1, db1, sems1)

                @pl.when(k + 1 < n_chunks)
                def _():
                    gissue(k + 1, rb1, db1, wb1, semg1)
                scale(wb0, rb0)
                sissue(rb0, db0, sems0)

            @pl.when(k % 2 == 1)
            def _():
                gwait(rb1, db1, wb1, semg1)
                swait(rb0, db0, sems0)

                @pl.when(k + 1 < n_chunks)
                def _():
                    gissue(k + 1, rb0, db0, wb0, semg0)
                scale(wb1, rb1)
                sissue(rb1, db1, sems1)
            return 0
        lax.fori_loop(0, n_chunks, chunk, 0)
        if (n_chunks - 1) % 2 == 0:
            swait(rb0, db0, sems0)
        else:
            swait(rb1, db1, sems1)
        plsc.subcore_barrier()

        for t in range(rows_per_tile // ZR):
            r0 = sid * rows_per_tile + t * ZR
            pltpu.sync_copy(acc_sh.at[pl.ds(r0, ZR)],
                            out_hbm.at[cid, pl.ds(r0, ZR)])

    return corr_sum


def _make_cause_max(n_dst, d2, n_edges):
    NC, NS = 2, 16
    NW = NC * NS
    RT = ((-(-n_dst // NW)) + 7) // 8 * 8
    n_pad = RT * NW
    ACC_R = RT + 8
    DUMMY = RT
    CH = 6400
    n_chunks = n_edges // CH
    assert n_edges % CH == 0
    SS = 2016
    BL = 96
    FLUSH_AT = SS - 256
    mesh = plsc.VectorSubcoreMesh(core_axis_name="c", subcore_axis_name="s")

    @functools.partial(
        pl.kernel, mesh=mesh,
        out_type=jax.ShapeDtypeStruct((n_pad, d2), jnp.float32),
        scratch_types=[
            pltpu.VMEM((CH,), jnp.int32),
            pltpu.VMEM((CH,), jnp.int32),
            pltpu.VMEM((CH,), jnp.float32),
            pltpu.VMEM((SS,), jnp.int32),
            pltpu.VMEM((SS,), jnp.float32),
            pltpu.VMEM((SS,), jnp.int32),
            pltpu.VMEM((BL, d2), jnp.float32),
            pltpu.VMEM((BL, d2), jnp.float32),
            pltpu.VMEM((ACC_R, d2), jnp.float32),
            pltpu.SemaphoreType.DMA,
            pltpu.SemaphoreType.DMA,
        ],
    )
    def cause_max(src_hbm, dst_hbm, w_hbm, x01_hbm, out_hbm,
                  srcc_v, dstc_v, wc_v, sstag, wstag, dstag,
                  rb0, rb1, acc_v, sem0, sem1):
        cid = lax.axis_index("c")
        sid = lax.axis_index("s")
        wid = sid * NC + cid
        lo = wid * RT

        def initrow(i, _):
            ninf16 = jnp.full((N_LANES,), -jnp.inf, jnp.float32)
            for j in range(d2 // N_LANES):
                acc_v[i, pl.ds(j * N_LANES, N_LANES)] = ninf16
            return 0
        lax.fori_loop(0, ACC_R, initrow, 0)

        def dummy_fill(g, _):
            sl = pl.ds(g * N_LANES, N_LANES)
            sstag[sl] = jnp.zeros((N_LANES,), jnp.int32)
            wstag[sl] = jnp.zeros((N_LANES,), jnp.float32)
            dstag[sl] = jnp.full((N_LANES,), DUMMY, jnp.int32)
            return 0
        lax.fori_loop(0, SS // N_LANES, dummy_fill, 0)

        def issue(b, rb, sem):
            return pltpu.async_copy(
                x01_hbm.at[sstag.at[pl.ds(b * BL, BL)]], rb, sem)

        def process(rb, g0):
            def grp(g, _):
                w16 = wstag[pl.ds(g0 * BL + g * N_LANES, N_LANES)]
                d16 = dstag[pl.ds(g0 * BL + g * N_LANES, N_LANES)]
                for r in range(N_LANES):
                    i = g * N_LANES + r
                    wb = jnp.full((N_LANES,), w16[r], jnp.float32)
                    dr = d16[r]
                    for j in range(d2 // N_LANES):
                        sl = pl.ds(j * N_LANES, N_LANES)
                        acc_v[dr, sl] = jnp.maximum(acc_v[dr, sl],
                                                    rb[i, sl] * wb)
                return 0
            lax.fori_loop(0, BL // N_LANES, grp, 0)

        def flush(ptr):
            nb = (ptr + BL - 1) // BL
            issue(0, rb0, sem0)

            def blk(b, _):
                p = b % 2

                @pl.when(p == 0)
                def _():
                    pltpu.make_async_copy(
                        x01_hbm.at[sstag.at[pl.ds(0, BL)]], rb0, sem0).wait()

                    @pl.when(b + 1 < nb)
                    def _():
                        issue(b + 1, rb1, sem1)
                    process(rb0, b)

                @pl.when(p == 1)
                def _():
                    pltpu.make_async_copy(
                        x01_hbm.at[sstag.at[pl.ds(0, BL)]], rb1, sem1).wait()

                    @pl.when(b + 1 < nb)
                    def _():
                        issue(b + 1, rb0, sem0)
                    process(rb1, b)
                return 0
            lax.fori_loop(0, nb, blk, 0)

        def grp_scan(g, ptr):
            sl = pl.ds(g * N_LANES, N_LANES)
            d16 = dstc_v[sl]
            lane = lax.iota(jnp.int32, N_LANES)
            one = jnp.full((N_LANES,), 1, jnp.int32)
            zero = jnp.full((N_LANES,), 0, jnp.int32)
            lo16 = jnp.full((N_LANES,), lo, jnp.int32)
            hi16 = jnp.full((N_LANES,), lo + RT, jnp.int32)
            m = (d16 >= lo16) & (d16 < hi16)
            pc = jnp.where(m, one, zero)
            for st in (1, 2, 4, 8):
                idx = jnp.maximum(lane - st, 0)
                sh = pc.at[idx].get(mode='promise_in_bounds')
                pc = pc + jnp.where(lane >= st, sh, zero)
            cnt = pc[N_LANES - 1]

            @pl.when(cnt > 0)
            def _():
                s16 = srcc_v[sl]
                w16 = wc_v[sl]
                target = lane + one
                pos = zero
                for st in (8, 4, 2, 1):
                    probe = pos + jnp.full((N_LANES,), st - 1, jnp.int32)
                    v = pc.at[probe].get(mode='promise_in_bounds')
                    pos = jnp.where(
                        v < target,
                        pos + jnp.full((N_LANES,), st, jnp.int32), pos)
                cnt16 = jnp.full((N_LANES,), cnt, jnp.int32)
                valid = lane < cnt16
                sg = s16.at[pos].get(mode='promise_in_bounds')
                wg = w16.at[pos].get(mode='promise_in_bounds')
                dg = d16.at[pos].get(mode='promise_in_bounds')
                psl = pl.ds(ptr, N_LANES)
                sstag[psl] = jnp.where(valid, sg, zero)
                wstag[psl] = jnp.where(valid, wg,
                                       jnp.full((N_LANES,), 0.0, jnp.float32))
                dstag[psl] = jnp.where(valid, dg - lo16,
                                       jnp.full((N_LANES,), DUMMY, jnp.int32))
            return ptr + cnt

        def subchunk(t, ptr):
            def gs(g, p):
                return grp_scan(t * 16 + g, p)
            ptr = lax.fori_loop(0, 16, gs, ptr, unroll=16)
            do = ptr >= FLUSH_AT

            @pl.when(do)
            def _():
                flush(ptr)
            return jnp.where(do, 0, ptr)

        def chunk(k, ptr):
            base = k * CH
            pltpu.sync_copy(src_hbm.at[pl.ds(base, CH)], srcc_v)
            pltpu.sync_copy(dst_hbm.at[pl.ds(base, CH)], dstc_v)
            pltpu.sync_copy(w_hbm.at[pl.ds(base, CH)], wc_v)
            return lax.fori_loop(0, CH // 256, subchunk, ptr)

        ptr = lax.fori_loop(0, n_chunks, chunk, 0)

        @pl.when(ptr > 0)
        def _():
            flush(ptr)

        pltpu.sync_copy(acc_v.at[pl.ds(0, RT)], out_hbm.at[pl.ds(lo, RT)])

    return cause_max


def kernel(x_metric, x_alert, edge_index_corr, edge_weight_corr,
           edge_index_cause, edge_weight_cause,
           Wr_c0, br_c0, Wroot_c0, Wr_a0, br_a0, Wroot_a0,
           Wr_c1, br_c1, Wroot_c1, Wr_a1, br_a1, Wroot_a1):
    n_m, d = x_metric.shape
    n_a = x_alert.shape[0]
    e_c = edge_index_corr.shape[1]

    src_c, dst_c = edge_index_corr[0], edge_index_corr[1]
    src_a, dst_a = edge_index_cause[0], edge_index_cause[1]

    agg_c = _make_corr_sum(n_m, d, e_c)(src_c, dst_c, edge_weight_corr,
                                        x_metric)
    m0 = _fused_layer(agg_c[:, :n_m], x_metric, Wr_c0, br_c0, Wroot_c0)

    e_a = edge_index_cause.shape[1]
    x01 = jnp.concatenate([x_metric, m0], axis=1)
    out01 = _make_cause_max(n_a, 2 * d, e_a)(
        src_a, dst_a, edge_weight_cause, x01)
    a0 = _fused_layer(out01[None, :n_a, :d], x_alert, Wr_a0, br_a0, Wroot_a0,
                      finite_fix=True)
    a1 = _fused_layer(out01[None, :n_a, d:], a0, Wr_a1, br_a1, Wroot_a1,
                      finite_fix=True)
    return a1
